# Initial kernel scaffold; baseline (speedup 1.0000x reference)
import functools

import jax
import jax.numpy as jnp
from jax import lax
from jax.experimental import pallas as pl
from jax.experimental.pallas import tpu as pltpu
from jax.experimental.pallas import tpu_sc as plsc

H = 1024
W = 1024
D = 64
NCOL = 10
L = 16
NC, NS = 2, 16
NW = NC * NS
ROWS_PER_W = H // NW
CHUNK = 128
NCHUNK = W // CHUNK


def _table_body(cpad_ref, x_ref, t_ref):
    for c in range(NCOL + 1):
        t_ref[c] = x_ref[...] + cpad_ref[c, :][None, :]


def _build_table(cpad, x_vec):
    return pl.pallas_call(
        _table_body,
        out_shape=jax.ShapeDtypeStruct((NCOL + 1, W, D), jnp.float32),
    )(cpad, x_vec)


_mesh = plsc.VectorSubcoreMesh(
    core_axis_name="c", subcore_axis_name="s", num_cores=NC, num_subcores=NS
)


@functools.partial(
    pl.kernel,
    out_type=jax.ShapeDtypeStruct((H * W, D), jnp.float32),
    mesh=_mesh,
    scratch_types=[
        pltpu.VMEM((W,), jnp.int32),
        pltpu.VMEM((NCHUNK, CHUNK), jnp.int32),
        pltpu.VMEM((W, D), jnp.float32),
        pltpu.VMEM((D,), jnp.float32),
        pltpu.SemaphoreType.DMA,
    ],
)
def _sc_embed(table_hbm, grid_hbm, y_hbm, out_hbm, gbuf, idxbuf, rowsbuf, ybuf, sem):
    wid = lax.axis_index("s") * NC + lax.axis_index("c")
    row0 = wid * ROWS_PER_W
    lanes = lax.iota(jnp.int32, L)

    def do_row(r, carry):
        i = row0 + r
        pltpu.sync_copy(grid_hbm.at[i], gbuf)
        pltpu.sync_copy(y_hbm.at[i], ybuf)

        def mk_idx(t, c2):
            g = gbuf[pl.ds(t * L, L)]
            gc = jnp.clip(g, -1, NCOL - 1)
            idx = (gc + 1) * W + (t * L + lanes)
            idxbuf[t // (CHUNK // L), pl.ds((t % (CHUNK // L)) * L, L)] = idx
            return c2

        lax.fori_loop(0, W // L, mk_idx, 0, unroll=4)

        for k in range(NCHUNK):
            pltpu.async_copy(
                table_hbm.at[idxbuf.at[k]],
                rowsbuf.at[pl.ds(k * CHUNK, CHUNK)],
                sem,
            ).wait()

        y0 = ybuf[pl.ds(0, L)]
        y1 = ybuf[pl.ds(L, L)]
        y2 = ybuf[pl.ds(2 * L, L)]
        y3 = ybuf[pl.ds(3 * L, L)]

        def add_y(p, c2):
            plsc.addupdate(rowsbuf.at[p, pl.ds(0, L)], y0)
            plsc.addupdate(rowsbuf.at[p, pl.ds(L, L)], y1)
            plsc.addupdate(rowsbuf.at[p, pl.ds(2 * L, L)], y2)
            plsc.addupdate(rowsbuf.at[p, pl.ds(3 * L, L)], y3)
            return c2

        lax.fori_loop(0, W, add_y, 0, unroll=4)

        pltpu.sync_copy(rowsbuf, out_hbm.at[pl.ds(i * W, W)])
        return carry

    lax.fori_loop(0, ROWS_PER_W, do_row, 0)


def kernel(grid, colour_emb, x_emb, y_emb):
    cpad = jnp.concatenate(
        [jnp.zeros((1, D), jnp.float32), colour_emb.astype(jnp.float32)], axis=0
    )
    table = _build_table(cpad, x_emb[:W].astype(jnp.float32))
    table_flat = table.reshape((NCOL + 1) * W, D)
    out = _sc_embed(table_flat, grid, y_emb[:H].astype(jnp.float32))
    return out.reshape(H, W, D)


# speedup vs baseline: 3.5644x; 3.5644x over previous
import functools

import jax
import jax.numpy as jnp
from jax import lax
from jax.experimental import pallas as pl
from jax.experimental.pallas import tpu as pltpu
from jax.experimental.pallas import tpu_sc as plsc

H = 1024
W = 1024
D = 64
NCOL = 10
L = 16
NC, NS = 2, 16
NW = NC * NS
ROWS_PER_W = H // NW
CHUNK = 128
NCHUNK = W // CHUNK


def _table_body(cpad_ref, x_ref, t_ref):
    for c in range(NCOL + 1):
        t_ref[c] = x_ref[...] + cpad_ref[c, :][None, :]


def _build_table(cpad, x_vec):
    return pl.pallas_call(
        _table_body,
        out_shape=jax.ShapeDtypeStruct((NCOL + 1, W, D), jnp.float32),
    )(cpad, x_vec)


_mesh = plsc.VectorSubcoreMesh(
    core_axis_name="c", subcore_axis_name="s", num_cores=NC, num_subcores=NS
)


@functools.partial(
    pl.kernel,
    out_type=jax.ShapeDtypeStruct((H * W, D), jnp.float32),
    mesh=_mesh,
    compiler_params=pltpu.CompilerParams(use_tc_tiling_on_sc=False),
    scratch_types=[
        pltpu.VMEM((W,), jnp.int32),
        pltpu.VMEM((NCHUNK, CHUNK), jnp.int32),
        pltpu.VMEM((W, D), jnp.float32),
        pltpu.VMEM((D,), jnp.float32),
        pltpu.SemaphoreType.DMA,
    ],
)
def _sc_embed(table_hbm, grid_hbm, y_hbm, out_hbm, gbuf, idxbuf, rowsbuf, ybuf, sem):
    wid = lax.axis_index("s") * NC + lax.axis_index("c")
    row0 = wid * ROWS_PER_W
    lanes = lax.iota(jnp.int32, L)

    def do_row(r, carry):
        i = row0 + r
        pltpu.sync_copy(grid_hbm.at[i], gbuf)
        pltpu.sync_copy(y_hbm.at[i], ybuf)

        def mk_idx(t, c2):
            g = gbuf[pl.ds(t * L, L)]
            gc = jnp.clip(g, -1, NCOL - 1)
            idx = (gc + 1) * W + (t * L + lanes)
            idxbuf[t // (CHUNK // L), pl.ds((t % (CHUNK // L)) * L, L)] = idx
            return c2

        lax.fori_loop(0, W // L, mk_idx, 0, unroll=4)

        for k in range(NCHUNK):
            pltpu.async_copy(
                table_hbm.at[idxbuf.at[k]],
                rowsbuf.at[pl.ds(k * CHUNK, CHUNK)],
                sem,
            ).wait()

        y0 = ybuf[pl.ds(0, L)]
        y1 = ybuf[pl.ds(L, L)]
        y2 = ybuf[pl.ds(2 * L, L)]
        y3 = ybuf[pl.ds(3 * L, L)]

        def add_y(p, c2):
            plsc.addupdate(rowsbuf.at[p, pl.ds(0, L)], y0)
            plsc.addupdate(rowsbuf.at[p, pl.ds(L, L)], y1)
            plsc.addupdate(rowsbuf.at[p, pl.ds(2 * L, L)], y2)
            plsc.addupdate(rowsbuf.at[p, pl.ds(3 * L, L)], y3)
            return c2

        lax.fori_loop(0, W, add_y, 0, unroll=4)

        pltpu.sync_copy(rowsbuf, out_hbm.at[pl.ds(i * W, W)])
        return carry

    lax.fori_loop(0, ROWS_PER_W, do_row, 0)


def kernel(grid, colour_emb, x_emb, y_emb):
    cpad = jnp.concatenate(
        [jnp.zeros((1, D), jnp.float32), colour_emb.astype(jnp.float32)], axis=0
    )
    table = _build_table(cpad, x_emb[:W].astype(jnp.float32))
    table_flat = table.reshape((NCOL + 1) * W, D)
    out = _sc_embed(table_flat, grid, y_emb[:H].astype(jnp.float32))
    return out.reshape(H, W, D)


# speedup vs baseline: 4.4785x; 1.2564x over previous
import functools

import jax
import jax.numpy as jnp
from jax import lax
from jax.experimental import pallas as pl
from jax.experimental.pallas import tpu as pltpu
from jax.experimental.pallas import tpu_sc as plsc

H = 1024
W = 1024
D = 64
NCOL = 10
L = 16
NC, NS = 2, 16
NW = NC * NS
ROWS_PER_W = H // NW
CHUNK = 128
NCHUNK = W // CHUNK


def _table_body(cpad_ref, x_ref, t_ref):
    for c in range(NCOL + 1):
        t_ref[c] = x_ref[...] + cpad_ref[c, :][None, :]


def _build_table(cpad, x_vec):
    return pl.pallas_call(
        _table_body,
        out_shape=jax.ShapeDtypeStruct((NCOL + 1, W, D), jnp.float32),
    )(cpad, x_vec)


_mesh = plsc.VectorSubcoreMesh(
    core_axis_name="c", subcore_axis_name="s", num_cores=NC, num_subcores=NS
)


HALF = W // 2
KPB = NCHUNK // 2


@functools.partial(
    pl.kernel,
    out_type=jax.ShapeDtypeStruct((H * W, D), jnp.float32),
    mesh=_mesh,
    compiler_params=pltpu.CompilerParams(use_tc_tiling_on_sc=False),
    scratch_types=[
        pltpu.VMEM((2, W), jnp.int32),
        pltpu.VMEM((2, D), jnp.float32),
        pltpu.VMEM((NCHUNK, CHUNK), jnp.int32),
        pltpu.VMEM((HALF, D), jnp.float32),
        pltpu.VMEM((HALF, D), jnp.float32),
        pltpu.SemaphoreType.DMA,
        pltpu.SemaphoreType.DMA,
        pltpu.SemaphoreType.DMA,
        pltpu.SemaphoreType.DMA,
        pltpu.SemaphoreType.DMA,
    ],
)
def _sc_embed(table_hbm, grid_hbm, y_hbm, out_hbm,
              gbuf, ybuf, idxbuf, rows0, rows1,
              gsem0, gsem1, osem0, osem1, psem):
    wid = lax.axis_index("s") * NC + lax.axis_index("c")
    row0 = wid * ROWS_PER_W
    lanes = lax.iota(jnp.int32, L)

    pltpu.sync_copy(grid_hbm.at[row0], gbuf.at[0])
    pltpu.sync_copy(y_hbm.at[row0], ybuf.at[0])

    def do_row(r, carry):
        i = row0 + r
        pp = lax.rem(r, 2)
        np_ = lax.rem(r + 1, 2)

        @pl.when(r < ROWS_PER_W - 1)
        def _prefetch():
            pltpu.async_copy(grid_hbm.at[i + 1], gbuf.at[np_], psem)
            pltpu.async_copy(y_hbm.at[i + 1], ybuf.at[np_], psem)

        def mk_idx(t, c2):
            g = gbuf[pp, pl.ds(t * L, L)]
            gc = jnp.clip(g, -1, NCOL - 1)
            idx = (gc + 1) * W + (t * L + lanes)
            idxbuf[t // (CHUNK // L), pl.ds((t % (CHUNK // L)) * L, L)] = idx
            return c2

        lax.fori_loop(0, W // L, mk_idx, 0, unroll=4)

        @pl.when(r > 0)
        def _drain_prev_scatters():
            base_prev = (i - 1) * W
            pltpu.make_async_copy(
                rows0, out_hbm.at[pl.ds(base_prev, HALF)], osem0).wait()
            pltpu.make_async_copy(
                rows1, out_hbm.at[pl.ds(base_prev + HALF, HALF)], osem1).wait()

        for k in range(KPB):
            pltpu.async_copy(
                table_hbm.at[idxbuf.at[k]],
                rows0.at[pl.ds(k * CHUNK, CHUNK)], gsem0)
        for k in range(KPB):
            pltpu.async_copy(
                table_hbm.at[idxbuf.at[KPB + k]],
                rows1.at[pl.ds(k * CHUNK, CHUNK)], gsem1)

        y0 = ybuf[pp, pl.ds(0, L)]
        y1 = ybuf[pp, pl.ds(L, L)]
        y2 = ybuf[pp, pl.ds(2 * L, L)]
        y3 = ybuf[pp, pl.ds(3 * L, L)]

        for k in range(KPB):
            pltpu.make_async_copy(
                table_hbm.at[idxbuf.at[k]],
                rows0.at[pl.ds(k * CHUNK, CHUNK)], gsem0).wait()

        def add_y0(p, c2):
            plsc.addupdate(rows0.at[p, pl.ds(0, L)], y0)
            plsc.addupdate(rows0.at[p, pl.ds(L, L)], y1)
            plsc.addupdate(rows0.at[p, pl.ds(2 * L, L)], y2)
            plsc.addupdate(rows0.at[p, pl.ds(3 * L, L)], y3)
            return c2

        lax.fori_loop(0, HALF, add_y0, 0, unroll=4)
        pltpu.async_copy(rows0, out_hbm.at[pl.ds(i * W, HALF)], osem0)

        for k in range(KPB):
            pltpu.make_async_copy(
                table_hbm.at[idxbuf.at[KPB + k]],
                rows1.at[pl.ds(k * CHUNK, CHUNK)], gsem1).wait()

        def add_y1(p, c2):
            plsc.addupdate(rows1.at[p, pl.ds(0, L)], y0)
            plsc.addupdate(rows1.at[p, pl.ds(L, L)], y1)
            plsc.addupdate(rows1.at[p, pl.ds(2 * L, L)], y2)
            plsc.addupdate(rows1.at[p, pl.ds(3 * L, L)], y3)
            return c2

        lax.fori_loop(0, HALF, add_y1, 0, unroll=4)
        pltpu.async_copy(rows1, out_hbm.at[pl.ds(i * W + HALF, HALF)], osem1)

        @pl.when(r < ROWS_PER_W - 1)
        def _drain_prefetch():
            pltpu.make_async_copy(grid_hbm.at[i + 1], gbuf.at[np_], psem).wait()
            pltpu.make_async_copy(y_hbm.at[i + 1], ybuf.at[np_], psem).wait()

        return carry

    lax.fori_loop(0, ROWS_PER_W, do_row, 0)

    base_last = (row0 + ROWS_PER_W - 1) * W
    pltpu.make_async_copy(
        rows0, out_hbm.at[pl.ds(base_last, HALF)], osem0).wait()
    pltpu.make_async_copy(
        rows1, out_hbm.at[pl.ds(base_last + HALF, HALF)], osem1).wait()


def kernel(grid, colour_emb, x_emb, y_emb):
    cpad = jnp.concatenate(
        [jnp.zeros((1, D), jnp.float32), colour_emb.astype(jnp.float32)], axis=0
    )
    table = _build_table(cpad, x_emb[:W].astype(jnp.float32))
    table_flat = table.reshape((NCOL + 1) * W, D)
    out = _sc_embed(table_flat, grid, y_emb[:H].astype(jnp.float32))
    return out.reshape(H, W, D)


# speedup vs baseline: 4.6609x; 1.0407x over previous
import functools

import jax
import jax.numpy as jnp
from jax import lax
from jax.experimental import pallas as pl
from jax.experimental.pallas import tpu as pltpu
from jax.experimental.pallas import tpu_sc as plsc

H = 1024
W = 1024
D = 64
NCOL = 10
L = 16
NC, NS = 2, 16
NW = NC * NS
ROWS_PER_W = H // NW
CHUNK = 128
NCHUNK = W // CHUNK


def _table_body(cpad_ref, x_ref, t_ref):
    for c in range(NCOL + 1):
        t_ref[c] = x_ref[...] + cpad_ref[c, :][None, :]


def _build_table(cpad, x_vec):
    return pl.pallas_call(
        _table_body,
        out_shape=jax.ShapeDtypeStruct((NCOL + 1, W, D), jnp.float32),
    )(cpad, x_vec)


_mesh = plsc.VectorSubcoreMesh(
    core_axis_name="c", subcore_axis_name="s", num_cores=NC, num_subcores=NS
)


HALF = W // 2
KPB = NCHUNK // 2


@functools.partial(
    pl.kernel,
    out_type=jax.ShapeDtypeStruct((H * W, D), jnp.float32),
    mesh=_mesh,
    compiler_params=pltpu.CompilerParams(use_tc_tiling_on_sc=False),
    scratch_types=[
        pltpu.VMEM((2, W), jnp.int32),
        pltpu.VMEM((2, D), jnp.float32),
        pltpu.VMEM((NCHUNK, CHUNK), jnp.int32),
        pltpu.VMEM((HALF, D), jnp.float32),
        pltpu.VMEM((HALF, D), jnp.float32),
        pltpu.VMEM_SHARED(((NCOL + 1) * W, D), jnp.float32),
        pltpu.SemaphoreType.DMA,
        pltpu.SemaphoreType.DMA,
        pltpu.SemaphoreType.DMA,
        pltpu.SemaphoreType.DMA,
        pltpu.SemaphoreType.DMA,
    ],
)
def _sc_embed(table_hbm, grid_hbm, y_hbm, out_hbm,
              gbuf, ybuf, idxbuf, rows0, rows1, tbl,
              gsem0, gsem1, osem0, osem1, psem):
    wid = lax.axis_index("s") * NC + lax.axis_index("c")
    row0 = wid * ROWS_PER_W
    lanes = lax.iota(jnp.int32, L)

    @pl.when(lax.axis_index("s") == 0)
    def _stage_table():
        pltpu.sync_copy(table_hbm, tbl)

    plsc.subcore_barrier()

    pltpu.sync_copy(grid_hbm.at[row0], gbuf.at[0])
    pltpu.sync_copy(y_hbm.at[row0], ybuf.at[0])

    def do_row(r, carry):
        i = row0 + r
        pp = lax.rem(r, 2)
        np_ = lax.rem(r + 1, 2)

        @pl.when(r < ROWS_PER_W - 1)
        def _prefetch():
            pltpu.async_copy(grid_hbm.at[i + 1], gbuf.at[np_], psem)
            pltpu.async_copy(y_hbm.at[i + 1], ybuf.at[np_], psem)

        def mk_idx(t, c2):
            g = gbuf[pp, pl.ds(t * L, L)]
            gc = jnp.clip(g, -1, NCOL - 1)
            idx = (gc + 1) * W + (t * L + lanes)
            idxbuf[t // (CHUNK // L), pl.ds((t % (CHUNK // L)) * L, L)] = idx
            return c2

        lax.fori_loop(0, W // L, mk_idx, 0, unroll=4)

        @pl.when(r > 0)
        def _drain_prev_scatters():
            base_prev = (i - 1) * W
            pltpu.make_async_copy(
                rows0, out_hbm.at[pl.ds(base_prev, HALF)], osem0).wait()
            pltpu.make_async_copy(
                rows1, out_hbm.at[pl.ds(base_prev + HALF, HALF)], osem1).wait()

        for k in range(KPB):
            pltpu.async_copy(
                tbl.at[idxbuf.at[k]],
                rows0.at[pl.ds(k * CHUNK, CHUNK)], gsem0)
        for k in range(KPB):
            pltpu.async_copy(
                tbl.at[idxbuf.at[KPB + k]],
                rows1.at[pl.ds(k * CHUNK, CHUNK)], gsem1)

        y0 = ybuf[pp, pl.ds(0, L)]
        y1 = ybuf[pp, pl.ds(L, L)]
        y2 = ybuf[pp, pl.ds(2 * L, L)]
        y3 = ybuf[pp, pl.ds(3 * L, L)]

        for k in range(KPB):
            pltpu.make_async_copy(
                tbl.at[idxbuf.at[k]],
                rows0.at[pl.ds(k * CHUNK, CHUNK)], gsem0).wait()

        def add_y0(p, c2):
            plsc.addupdate(rows0.at[p, pl.ds(0, L)], y0)
            plsc.addupdate(rows0.at[p, pl.ds(L, L)], y1)
            plsc.addupdate(rows0.at[p, pl.ds(2 * L, L)], y2)
            plsc.addupdate(rows0.at[p, pl.ds(3 * L, L)], y3)
            return c2

        lax.fori_loop(0, HALF, add_y0, 0, unroll=4)
        pltpu.async_copy(rows0, out_hbm.at[pl.ds(i * W, HALF)], osem0)

        for k in range(KPB):
            pltpu.make_async_copy(
                tbl.at[idxbuf.at[KPB + k]],
                rows1.at[pl.ds(k * CHUNK, CHUNK)], gsem1).wait()

        def add_y1(p, c2):
            plsc.addupdate(rows1.at[p, pl.ds(0, L)], y0)
            plsc.addupdate(rows1.at[p, pl.ds(L, L)], y1)
            plsc.addupdate(rows1.at[p, pl.ds(2 * L, L)], y2)
            plsc.addupdate(rows1.at[p, pl.ds(3 * L, L)], y3)
            return c2

        lax.fori_loop(0, HALF, add_y1, 0, unroll=4)
        pltpu.async_copy(rows1, out_hbm.at[pl.ds(i * W + HALF, HALF)], osem1)

        @pl.when(r < ROWS_PER_W - 1)
        def _drain_prefetch():
            pltpu.make_async_copy(grid_hbm.at[i + 1], gbuf.at[np_], psem).wait()
            pltpu.make_async_copy(y_hbm.at[i + 1], ybuf.at[np_], psem).wait()

        return carry

    lax.fori_loop(0, ROWS_PER_W, do_row, 0)

    base_last = (row0 + ROWS_PER_W - 1) * W
    pltpu.make_async_copy(
        rows0, out_hbm.at[pl.ds(base_last, HALF)], osem0).wait()
    pltpu.make_async_copy(
        rows1, out_hbm.at[pl.ds(base_last + HALF, HALF)], osem1).wait()


def kernel(grid, colour_emb, x_emb, y_emb):
    cpad = jnp.concatenate(
        [jnp.zeros((1, D), jnp.float32), colour_emb.astype(jnp.float32)], axis=0
    )
    table = _build_table(cpad, x_emb[:W].astype(jnp.float32))
    table_flat = table.reshape((NCOL + 1) * W, D)
    out = _sc_embed(table_flat, grid, y_emb[:H].astype(jnp.float32))
    return out.reshape(H, W, D)


# speedup vs baseline: 4.6646x; 1.0008x over previous
import functools

import jax
import jax.numpy as jnp
from jax import lax
from jax.experimental import pallas as pl
from jax.experimental.pallas import tpu as pltpu
from jax.experimental.pallas import tpu_sc as plsc

H = 1024
W = 1024
D = 64
NCOL = 10
L = 16
NC, NS = 2, 16
NW = NC * NS
ROWS_PER_W = H // NW
CHUNK = 128
NCHUNK = W // CHUNK


def _table_body(cpad_ref, x_ref, t_ref):
    for c in range(NCOL + 1):
        t_ref[c] = x_ref[...] + cpad_ref[c, :][None, :]


def _build_table(cpad, x_vec):
    return pl.pallas_call(
        _table_body,
        out_shape=jax.ShapeDtypeStruct((NCOL + 1, W, D), jnp.float32),
    )(cpad, x_vec)


_mesh = plsc.VectorSubcoreMesh(
    core_axis_name="c", subcore_axis_name="s", num_cores=NC, num_subcores=NS
)


HALF = W // 2
KPB = NCHUNK // 2


@functools.partial(
    pl.kernel,
    out_type=jax.ShapeDtypeStruct((H * W, D), jnp.float32),
    mesh=_mesh,
    compiler_params=pltpu.CompilerParams(use_tc_tiling_on_sc=False),
    scratch_types=[
        pltpu.VMEM((2, W), jnp.int32),
        pltpu.VMEM((2, D), jnp.float32),
        pltpu.VMEM((NCHUNK, CHUNK), jnp.int32),
        pltpu.VMEM((HALF, D), jnp.float32),
        pltpu.VMEM((HALF, D), jnp.float32),
        pltpu.VMEM_SHARED(((NCOL + 1) * W, D), jnp.float32),
        pltpu.SemaphoreType.DMA,
        pltpu.SemaphoreType.DMA,
        pltpu.SemaphoreType.DMA,
        pltpu.SemaphoreType.DMA,
        pltpu.SemaphoreType.DMA,
    ],
)
def _sc_embed(table_hbm, grid_hbm, y_hbm, out_hbm,
              gbuf, ybuf, idxbuf, rows0, rows1, tbl,
              gsem0, gsem1, osem0, osem1, psem):
    wid = lax.axis_index("s") * NC + lax.axis_index("c")
    row0 = wid * ROWS_PER_W
    lanes = lax.iota(jnp.int32, L)

    @pl.when(lax.axis_index("s") == 0)
    def _stage_table():
        pltpu.sync_copy(table_hbm, tbl)

    plsc.subcore_barrier()

    pltpu.sync_copy(grid_hbm.at[row0], gbuf.at[0])
    pltpu.sync_copy(y_hbm.at[row0], ybuf.at[0])

    def do_row(r, carry):
        i = row0 + r
        pp = lax.rem(r, 2)
        np_ = lax.rem(r + 1, 2)

        @pl.when(r < ROWS_PER_W - 1)
        def _prefetch():
            pltpu.async_copy(grid_hbm.at[i + 1], gbuf.at[np_], psem)
            pltpu.async_copy(y_hbm.at[i + 1], ybuf.at[np_], psem)

        @plsc.parallel_loop(0, W // L, unroll=4)
        def mk_idx(t):
            g = gbuf[pp, pl.ds(t * L, L)]
            gc = jnp.clip(g, -1, NCOL - 1)
            idx = (gc + 1) * W + (t * L + lanes)
            idxbuf[t // (CHUNK // L), pl.ds((t % (CHUNK // L)) * L, L)] = idx

        @pl.when(r > 0)
        def _drain_prev_scatters():
            base_prev = (i - 1) * W
            pltpu.make_async_copy(
                rows0, out_hbm.at[pl.ds(base_prev, HALF)], osem0).wait()
            pltpu.make_async_copy(
                rows1, out_hbm.at[pl.ds(base_prev + HALF, HALF)], osem1).wait()

        for k in range(KPB):
            pltpu.async_copy(
                tbl.at[idxbuf.at[k]],
                rows0.at[pl.ds(k * CHUNK, CHUNK)], gsem0)
        for k in range(KPB):
            pltpu.async_copy(
                tbl.at[idxbuf.at[KPB + k]],
                rows1.at[pl.ds(k * CHUNK, CHUNK)], gsem1)

        y0 = ybuf[pp, pl.ds(0, L)]
        y1 = ybuf[pp, pl.ds(L, L)]
        y2 = ybuf[pp, pl.ds(2 * L, L)]
        y3 = ybuf[pp, pl.ds(3 * L, L)]

        for k in range(KPB):
            pltpu.make_async_copy(
                tbl.at[idxbuf.at[k]],
                rows0.at[pl.ds(k * CHUNK, CHUNK)], gsem0).wait()

        @plsc.parallel_loop(0, HALF, unroll=8)
        def add_y0(p):
            plsc.addupdate(rows0.at[p, pl.ds(0, L)], y0)
            plsc.addupdate(rows0.at[p, pl.ds(L, L)], y1)
            plsc.addupdate(rows0.at[p, pl.ds(2 * L, L)], y2)
            plsc.addupdate(rows0.at[p, pl.ds(3 * L, L)], y3)
        pltpu.async_copy(rows0, out_hbm.at[pl.ds(i * W, HALF)], osem0)

        for k in range(KPB):
            pltpu.make_async_copy(
                tbl.at[idxbuf.at[KPB + k]],
                rows1.at[pl.ds(k * CHUNK, CHUNK)], gsem1).wait()

        @plsc.parallel_loop(0, HALF, unroll=8)
        def add_y1(p):
            plsc.addupdate(rows1.at[p, pl.ds(0, L)], y0)
            plsc.addupdate(rows1.at[p, pl.ds(L, L)], y1)
            plsc.addupdate(rows1.at[p, pl.ds(2 * L, L)], y2)
            plsc.addupdate(rows1.at[p, pl.ds(3 * L, L)], y3)
        pltpu.async_copy(rows1, out_hbm.at[pl.ds(i * W + HALF, HALF)], osem1)

        @pl.when(r < ROWS_PER_W - 1)
        def _drain_prefetch():
            pltpu.make_async_copy(grid_hbm.at[i + 1], gbuf.at[np_], psem).wait()
            pltpu.make_async_copy(y_hbm.at[i + 1], ybuf.at[np_], psem).wait()

        return carry

    lax.fori_loop(0, ROWS_PER_W, do_row, 0)

    base_last = (row0 + ROWS_PER_W - 1) * W
    pltpu.make_async_copy(
        rows0, out_hbm.at[pl.ds(base_last, HALF)], osem0).wait()
    pltpu.make_async_copy(
        rows1, out_hbm.at[pl.ds(base_last + HALF, HALF)], osem1).wait()


def kernel(grid, colour_emb, x_emb, y_emb):
    cpad = jnp.concatenate(
        [jnp.zeros((1, D), jnp.float32), colour_emb.astype(jnp.float32)], axis=0
    )
    table = _build_table(cpad, x_emb[:W].astype(jnp.float32))
    table_flat = table.reshape((NCOL + 1) * W, D)
    out = _sc_embed(table_flat, grid, y_emb[:H].astype(jnp.float32))
    return out.reshape(H, W, D)
